# baseline (device time: 12554 ns/iter reference)
import jax
import jax.numpy as jnp
from jax import lax
from jax.experimental import pallas as pl
from jax.experimental.pallas import tpu as pltpu

_CHUNK = 128


def kernel(x, dy, gamma):
    m, d = x.shape
    half = m // 2
    n_chunks = half // _CHUNK

    def body(x_hbm, dy_hbm, out_ref,
             xbuf, dybuf, comm_ref, xsems, dysems,
             send_sem1, recv_sem1, send_sem2, recv_sem2):
        my_x = lax.axis_index("x")
        my_y = lax.axis_index("y")
        x_partner = (1 - my_x, my_y)
        y_partner = (my_x, 1 - my_y)
        off = my_y * half

        def chunk_copies(i, slot):
            cp_x = pltpu.make_async_copy(
                x_hbm.at[pl.ds(off + i * _CHUNK, _CHUNK)],
                xbuf.at[slot], xsems.at[slot])
            cp_dy = pltpu.make_async_copy(
                dy_hbm.at[pl.ds(off + i * _CHUNK, _CHUNK)],
                dybuf.at[slot], dysems.at[slot])
            return cp_x, cp_dy

        cp = chunk_copies(0, 0)
        cp[0].start()
        cp[1].start()

        acc_dg = jnp.zeros((d,), jnp.float32)
        acc_db = jnp.zeros((d,), jnp.float32)
        for i in range(n_chunks):
            slot = i % 2
            wait_cp = chunk_copies(i, slot)
            if i + 1 < n_chunks:
                nxt = chunk_copies(i + 1, (i + 1) % 2)
                nxt[0].start()
                nxt[1].start()
            wait_cp[0].wait()
            wait_cp[1].wait()
            xv = xbuf[slot]
            dyv = dybuf[slot]
            mu = jnp.mean(xv, axis=1, keepdims=True)
            var = jnp.mean((xv - mu) * (xv - mu), axis=1, keepdims=True)
            xhat = (xv - mu) * lax.rsqrt(var + 1e-5)
            acc_dg = acc_dg + jnp.sum(dyv * xhat, axis=0)
            acc_db = acc_db + jnp.sum(dyv, axis=0)

        comm_ref[0, 0, :] = acc_dg
        comm_ref[0, 1, :] = acc_db

        barrier_sem = pltpu.get_barrier_semaphore()
        for nbr in (x_partner, y_partner):
            pl.semaphore_signal(
                barrier_sem, inc=1,
                device_id=nbr, device_id_type=pl.DeviceIdType.MESH,
            )
        pl.semaphore_wait(barrier_sem, 2)

        rdma1 = pltpu.make_async_remote_copy(
            src_ref=comm_ref.at[0], dst_ref=comm_ref.at[1],
            send_sem=send_sem1, recv_sem=recv_sem1,
            device_id=x_partner, device_id_type=pl.DeviceIdType.MESH,
        )
        rdma1.start()
        rdma1.wait()
        comm_ref[0, :, :] = comm_ref[0] + comm_ref[1]

        rdma2 = pltpu.make_async_remote_copy(
            src_ref=comm_ref.at[0], dst_ref=comm_ref.at[2],
            send_sem=send_sem2, recv_sem=recv_sem2,
            device_id=y_partner, device_id_type=pl.DeviceIdType.MESH,
        )
        rdma2.start()
        rdma2.wait()
        out_ref[:, :] = comm_ref[0] + comm_ref[2]

    return pl.pallas_call(
        body,
        out_shape=jax.ShapeDtypeStruct((2, d), jnp.float32),
        in_specs=[
            pl.BlockSpec(memory_space=pl.ANY),
            pl.BlockSpec(memory_space=pl.ANY),
        ],
        out_specs=pl.BlockSpec(memory_space=pltpu.VMEM),
        scratch_shapes=[
            pltpu.VMEM((2, _CHUNK, d), jnp.float32),
            pltpu.VMEM((2, _CHUNK, d), jnp.float32),
            pltpu.VMEM((3, 2, d), jnp.float32),
            pltpu.SemaphoreType.DMA((2,)),
            pltpu.SemaphoreType.DMA((2,)),
            pltpu.SemaphoreType.DMA,
            pltpu.SemaphoreType.DMA,
            pltpu.SemaphoreType.DMA,
            pltpu.SemaphoreType.DMA,
        ],
        compiler_params=pltpu.CompilerParams(collective_id=0),
    )(x, dy)


# device time: 11863 ns/iter; 1.0582x vs baseline; 1.0582x over previous
import jax
import jax.numpy as jnp
from jax import lax
from jax.experimental import pallas as pl
from jax.experimental.pallas import tpu as pltpu


def kernel(x, dy, gamma):
    m, d = x.shape
    half = m // 2

    def body(x_ref, dy_ref, out_ref, comm_ref,
             send_sem1, recv_sem1, send_sem2, recv_sem2):
        my_x = lax.axis_index("x")
        my_y = lax.axis_index("y")
        x_partner = (1 - my_x, my_y)
        y_partner = (my_x, 1 - my_y)
        off = my_y * half

        xv = x_ref[pl.ds(off, half), :]
        dyv = dy_ref[pl.ds(off, half), :]
        mu = jnp.mean(xv, axis=1, keepdims=True)
        var = jnp.mean((xv - mu) * (xv - mu), axis=1, keepdims=True)
        xhat = (xv - mu) * lax.rsqrt(var + 1e-5)
        comm_ref[0, 0, :] = jnp.sum(dyv * xhat, axis=0)
        comm_ref[0, 1, :] = jnp.sum(dyv, axis=0)

        barrier_sem = pltpu.get_barrier_semaphore()
        for nbr in (x_partner, y_partner):
            pl.semaphore_signal(
                barrier_sem, inc=1,
                device_id=nbr, device_id_type=pl.DeviceIdType.MESH,
            )
        pl.semaphore_wait(barrier_sem, 2)

        rdma1 = pltpu.make_async_remote_copy(
            src_ref=comm_ref.at[0], dst_ref=comm_ref.at[1],
            send_sem=send_sem1, recv_sem=recv_sem1,
            device_id=x_partner, device_id_type=pl.DeviceIdType.MESH,
        )
        rdma1.start()
        rdma1.wait()
        comm_ref[0, :, :] = comm_ref[0] + comm_ref[1]

        rdma2 = pltpu.make_async_remote_copy(
            src_ref=comm_ref.at[0], dst_ref=comm_ref.at[2],
            send_sem=send_sem2, recv_sem=recv_sem2,
            device_id=y_partner, device_id_type=pl.DeviceIdType.MESH,
        )
        rdma2.start()
        rdma2.wait()
        out_ref[:, :] = comm_ref[0] + comm_ref[2]

    return pl.pallas_call(
        body,
        out_shape=jax.ShapeDtypeStruct((2, d), jnp.float32),
        in_specs=[
            pl.BlockSpec(memory_space=pltpu.VMEM),
            pl.BlockSpec(memory_space=pltpu.VMEM),
        ],
        out_specs=pl.BlockSpec(memory_space=pltpu.VMEM),
        scratch_shapes=[
            pltpu.VMEM((3, 2, d), jnp.float32),
            pltpu.SemaphoreType.DMA,
            pltpu.SemaphoreType.DMA,
            pltpu.SemaphoreType.DMA,
            pltpu.SemaphoreType.DMA,
        ],
        compiler_params=pltpu.CompilerParams(collective_id=0),
    )(x, dy)


# device time: 10720 ns/iter; 1.1711x vs baseline; 1.1066x over previous
import jax
import jax.numpy as jnp
from jax import lax
from jax.experimental import pallas as pl
from jax.experimental.pallas import tpu as pltpu


def kernel(x, dy, gamma):
    m, d = x.shape
    half = m // 2

    def body(x_hbm, dy_hbm, out_ref, xbuf, dybuf, comm_ref,
             xcp_sem, dycp_sem, send_sems, recv_sems):
        my_x = lax.axis_index("x")
        my_y = lax.axis_index("y")
        peers = (
            (1 - my_x, my_y),
            (my_x, 1 - my_y),
            (1 - my_x, 1 - my_y),
        )
        off = my_y * half

        cp_x = pltpu.make_async_copy(
            x_hbm.at[pl.ds(off, half)], xbuf, xcp_sem)
        cp_dy = pltpu.make_async_copy(
            dy_hbm.at[pl.ds(off, half)], dybuf, dycp_sem)
        cp_x.start()
        cp_dy.start()

        barrier_sem = pltpu.get_barrier_semaphore()
        for nbr in peers:
            pl.semaphore_signal(
                barrier_sem, inc=1,
                device_id=nbr, device_id_type=pl.DeviceIdType.MESH,
            )

        cp_x.wait()
        cp_dy.wait()
        xv = xbuf[:, :]
        dyv = dybuf[:, :]
        mu = jnp.mean(xv, axis=1, keepdims=True)
        var = jnp.mean((xv - mu) * (xv - mu), axis=1, keepdims=True)
        xhat = (xv - mu) * lax.rsqrt(var + 1e-5)
        comm_ref[0, 0, :] = jnp.sum(dyv * xhat, axis=0)
        comm_ref[0, 1, :] = jnp.sum(dyv, axis=0)

        pl.semaphore_wait(barrier_sem, 3)

        rdmas = []
        for i, nbr in enumerate(peers):
            rdma = pltpu.make_async_remote_copy(
                src_ref=comm_ref.at[0], dst_ref=comm_ref.at[i + 1],
                send_sem=send_sems.at[i], recv_sem=recv_sems.at[i],
                device_id=nbr, device_id_type=pl.DeviceIdType.MESH,
            )
            rdma.start()
            rdmas.append(rdma)
        for rdma in rdmas:
            rdma.wait()

        out_ref[:, :] = (comm_ref[0] + comm_ref[1]) + (comm_ref[2] + comm_ref[3])

    return pl.pallas_call(
        body,
        out_shape=jax.ShapeDtypeStruct((2, d), jnp.float32),
        in_specs=[
            pl.BlockSpec(memory_space=pl.ANY),
            pl.BlockSpec(memory_space=pl.ANY),
        ],
        out_specs=pl.BlockSpec(memory_space=pltpu.VMEM),
        scratch_shapes=[
            pltpu.VMEM((half, d), jnp.float32),
            pltpu.VMEM((half, d), jnp.float32),
            pltpu.VMEM((4, 2, d), jnp.float32),
            pltpu.SemaphoreType.DMA,
            pltpu.SemaphoreType.DMA,
            pltpu.SemaphoreType.DMA((3,)),
            pltpu.SemaphoreType.DMA((3,)),
        ],
        compiler_params=pltpu.CompilerParams(collective_id=0),
    )(x, dy)


# device time: 9913 ns/iter; 1.2664x vs baseline; 1.0814x over previous
import jax
import jax.numpy as jnp
from jax import lax
from jax.experimental import pallas as pl
from jax.experimental.pallas import tpu as pltpu


def kernel(x, dy, gamma):
    m, d = x.shape

    def body(x_ref, dy_ref, out_ref, comm_ref, send_sem, recv_sem):
        my_x = lax.axis_index("x")
        my_y = lax.axis_index("y")
        partner = (1 - my_x, my_y)

        barrier_sem = pltpu.get_barrier_semaphore()
        pl.semaphore_signal(
            barrier_sem, inc=1,
            device_id=partner, device_id_type=pl.DeviceIdType.MESH,
        )

        xv = x_ref[:, :]
        dyv = dy_ref[:, :]
        mu = jnp.mean(xv, axis=1, keepdims=True)
        var = jnp.mean((xv - mu) * (xv - mu), axis=1, keepdims=True)
        xhat = (xv - mu) * lax.rsqrt(var + 1e-5)
        comm_ref[0, 0, :] = jnp.sum(dyv * xhat, axis=0)
        comm_ref[0, 1, :] = jnp.sum(dyv, axis=0)

        pl.semaphore_wait(barrier_sem, 1)

        rdma = pltpu.make_async_remote_copy(
            src_ref=comm_ref.at[0],
            dst_ref=comm_ref.at[1],
            send_sem=send_sem,
            recv_sem=recv_sem,
            device_id=partner,
            device_id_type=pl.DeviceIdType.MESH,
        )
        rdma.start()
        rdma.wait()

        out_ref[:, :] = comm_ref[0] + comm_ref[1]

    return pl.pallas_call(
        body,
        out_shape=jax.ShapeDtypeStruct((2, d), jnp.float32),
        in_specs=[
            pl.BlockSpec(memory_space=pltpu.VMEM),
            pl.BlockSpec(memory_space=pltpu.VMEM),
        ],
        out_specs=pl.BlockSpec(memory_space=pltpu.VMEM),
        scratch_shapes=[
            pltpu.VMEM((2, 2, d), jnp.float32),
            pltpu.SemaphoreType.DMA,
            pltpu.SemaphoreType.DMA,
        ],
        compiler_params=pltpu.CompilerParams(collective_id=0),
    )(x, dy)


# device time: 9267 ns/iter; 1.3547x vs baseline; 1.0697x over previous
import jax
import jax.numpy as jnp
from jax import lax
from jax.experimental import pallas as pl
from jax.experimental.pallas import tpu as pltpu


def kernel(x, dy, gamma):
    m, d = x.shape
    half = m // 2

    off = lax.axis_index("y") * half
    xh = lax.dynamic_slice(x, (off, 0), (half, d))
    dyh = lax.dynamic_slice(dy, (off, 0), (half, d))

    def body(x_ref, dy_ref, out_ref, comm_ref, send_sems, recv_sems):
        my_x = lax.axis_index("x")
        my_y = lax.axis_index("y")
        peers = (
            (1 - my_x, my_y),
            (my_x, 1 - my_y),
            (1 - my_x, 1 - my_y),
        )

        barrier_sem = pltpu.get_barrier_semaphore()
        for nbr in peers:
            pl.semaphore_signal(
                barrier_sem, inc=1,
                device_id=nbr, device_id_type=pl.DeviceIdType.MESH,
            )

        xv = x_ref[:, :]
        dyv = dy_ref[:, :]
        mu = jnp.mean(xv, axis=1, keepdims=True)
        var = jnp.mean((xv - mu) * (xv - mu), axis=1, keepdims=True)
        xhat = (xv - mu) * lax.rsqrt(var + 1e-5)
        comm_ref[0, 0, :] = jnp.sum(dyv * xhat, axis=0)
        comm_ref[0, 1, :] = jnp.sum(dyv, axis=0)

        pl.semaphore_wait(barrier_sem, 3)

        rdmas = []
        for i, nbr in enumerate(peers):
            rdma = pltpu.make_async_remote_copy(
                src_ref=comm_ref.at[0], dst_ref=comm_ref.at[i + 1],
                send_sem=send_sems.at[i], recv_sem=recv_sems.at[i],
                device_id=nbr, device_id_type=pl.DeviceIdType.MESH,
            )
            rdma.start()
            rdmas.append(rdma)
        for rdma in rdmas:
            rdma.wait_recv()
        out_ref[:, :] = (comm_ref[0] + comm_ref[1]) + (comm_ref[2] + comm_ref[3])
        for rdma in rdmas:
            rdma.wait_send()

    return pl.pallas_call(
        body,
        out_shape=jax.ShapeDtypeStruct((2, d), jnp.float32),
        in_specs=[
            pl.BlockSpec(memory_space=pltpu.VMEM),
            pl.BlockSpec(memory_space=pltpu.VMEM),
        ],
        out_specs=pl.BlockSpec(memory_space=pltpu.VMEM),
        scratch_shapes=[
            pltpu.VMEM((4, 2, d), jnp.float32),
            pltpu.SemaphoreType.DMA((3,)),
            pltpu.SemaphoreType.DMA((3,)),
        ],
        compiler_params=pltpu.CompilerParams(collective_id=0),
    )(xh, dyh)


# device time: 9251 ns/iter; 1.3570x vs baseline; 1.0017x over previous
import jax
import jax.numpy as jnp
from jax import lax
from jax.experimental import pallas as pl
from jax.experimental.pallas import tpu as pltpu


def kernel(x, dy, gamma):
    m, d = x.shape
    half = m // 2

    off = lax.axis_index("y") * half
    xh = pltpu.with_memory_space_constraint(
        lax.dynamic_slice(x, (off, 0), (half, d)), pltpu.MemorySpace.VMEM)
    dyh = pltpu.with_memory_space_constraint(
        lax.dynamic_slice(dy, (off, 0), (half, d)), pltpu.MemorySpace.VMEM)

    def body(x_ref, dy_ref, out_ref, comm_ref, send_sems, recv_sems):
        my_x = lax.axis_index("x")
        my_y = lax.axis_index("y")
        peers = (
            (1 - my_x, my_y),
            (my_x, 1 - my_y),
            (1 - my_x, 1 - my_y),
        )

        barrier_sem = pltpu.get_barrier_semaphore()
        for nbr in peers:
            pl.semaphore_signal(
                barrier_sem, inc=1,
                device_id=nbr, device_id_type=pl.DeviceIdType.MESH,
            )

        xv = x_ref[:, :]
        dyv = dy_ref[:, :]
        mu = jnp.mean(xv, axis=1, keepdims=True)
        var = jnp.mean((xv - mu) * (xv - mu), axis=1, keepdims=True)
        xhat = (xv - mu) * lax.rsqrt(var + 1e-5)
        comm_ref[0, 0, :] = jnp.sum(dyv * xhat, axis=0)
        comm_ref[0, 1, :] = jnp.sum(dyv, axis=0)

        pl.semaphore_wait(barrier_sem, 3)

        rdmas = []
        for i, nbr in enumerate(peers):
            rdma = pltpu.make_async_remote_copy(
                src_ref=comm_ref.at[0], dst_ref=comm_ref.at[i + 1],
                send_sem=send_sems.at[i], recv_sem=recv_sems.at[i],
                device_id=nbr, device_id_type=pl.DeviceIdType.MESH,
            )
            rdma.start()
            rdmas.append(rdma)
        for rdma in rdmas:
            rdma.wait_recv()
        out_ref[:, :] = (comm_ref[0] + comm_ref[1]) + (comm_ref[2] + comm_ref[3])
        for rdma in rdmas:
            rdma.wait_send()

    return pl.pallas_call(
        body,
        out_shape=jax.ShapeDtypeStruct((2, d), jnp.float32),
        in_specs=[
            pl.BlockSpec(memory_space=pltpu.VMEM),
            pl.BlockSpec(memory_space=pltpu.VMEM),
        ],
        out_specs=pl.BlockSpec(memory_space=pltpu.VMEM),
        scratch_shapes=[
            pltpu.VMEM((4, 2, d), jnp.float32),
            pltpu.SemaphoreType.DMA((3,)),
            pltpu.SemaphoreType.DMA((3,)),
        ],
        compiler_params=pltpu.CompilerParams(collective_id=0),
    )(xh, dyh)


# device time: 7995 ns/iter; 1.5702x vs baseline; 1.1571x over previous
import jax
import jax.numpy as jnp
from jax import lax
from jax.experimental import pallas as pl
from jax.experimental.pallas import tpu as pltpu


def kernel(x, dy, gamma):
    m, d = x.shape
    half = m // 2

    off = lax.axis_index("y") * half
    xh = lax.dynamic_slice(x, (off, 0), (half, d))
    dyh = lax.dynamic_slice(dy, (off, 0), (half, d))

    def body(x_ref, dy_ref, out_ref, comm_ref, send_sems, recv_sems):
        my_x = lax.axis_index("x")
        my_y = lax.axis_index("y")
        peers = (
            (1 - my_x, my_y),
            (my_x, 1 - my_y),
            (1 - my_x, 1 - my_y),
        )

        barrier_sem = pltpu.get_barrier_semaphore()
        for nbr in peers:
            pl.semaphore_signal(
                barrier_sem, inc=1,
                device_id=nbr, device_id_type=pl.DeviceIdType.MESH,
            )

        xv = x_ref[:, :]
        dyv = dy_ref[:, :]
        mu = jnp.mean(xv, axis=1, keepdims=True)
        var = jnp.mean((xv - mu) * (xv - mu), axis=1, keepdims=True)
        xhat = (xv - mu) * lax.rsqrt(var + 1e-5)
        comm_ref[0, 0, :] = jnp.sum(dyv * xhat, axis=0)
        comm_ref[0, 1, :] = jnp.sum(dyv, axis=0)

        pl.semaphore_wait(barrier_sem, 3)

        rdmas = []
        for i, nbr in enumerate(peers):
            rdma = pltpu.make_async_remote_copy(
                src_ref=comm_ref.at[0], dst_ref=comm_ref.at[i + 1],
                send_sem=send_sems.at[i], recv_sem=recv_sems.at[i],
                device_id=nbr, device_id_type=pl.DeviceIdType.MESH,
            )
            rdma.start()
            rdmas.append(rdma)
        for rdma in rdmas:
            rdma.wait_recv()
        out_ref[:, :] = (comm_ref[0] + comm_ref[1]) + (comm_ref[2] + comm_ref[3])
        for rdma in rdmas:
            rdma.wait_send()

    return pl.pallas_call(
        body,
        out_shape=jax.ShapeDtypeStruct((2, d), jnp.float32),
        in_specs=[
            pl.BlockSpec(memory_space=pltpu.VMEM),
            pl.BlockSpec(memory_space=pltpu.VMEM),
        ],
        out_specs=pl.BlockSpec(memory_space=pltpu.VMEM),
        scratch_shapes=[
            pltpu.VMEM((4, 2, d), jnp.float32),
            pltpu.SemaphoreType.DMA((3,)),
            pltpu.SemaphoreType.DMA((3,)),
        ],
        compiler_params=pltpu.CompilerParams(
            collective_id=0,
            allow_input_fusion=(True, True),
        ),
    )(xh, dyh)
